# baseline (speedup 1.0000x reference)
import jax
import jax.numpy as jnp
from jax.experimental import pallas as pl
from jax.experimental.pallas import tpu as pltpu

DIM = 1024
HID = 1024
E = 16
T = 16
LIMIT = 7.0
EPS = 1e-5

NC1 = 4
NC2 = 2
R1 = HID // NC1
R2 = DIM // NC2


def _moe(x_ref, nw_ref, gw_ref, gb_ref, b1g_ref, b1l_ref, b2_ref,
         w1_hbm, w2_hbm, out_ref, xn_ref, wd_ref, w1_buf, w2_buf,
         sem1, sem2):
    i = pl.program_id(0)

    def start(src, buf):
        for c in range(NC1):
            pltpu.make_async_copy(
                w1_hbm.at[src, pl.ds(c * R1, R1), :],
                w1_buf.at[buf, pl.ds(c * R1, R1), :],
                sem1.at[buf, c]).start()
        for c in range(NC2):
            pltpu.make_async_copy(
                w2_hbm.at[src, pl.ds(c * R2, R2), :],
                w2_buf.at[buf, pl.ds(c * R2, R2), :],
                sem2.at[buf, c]).start()

    def wait(buf):
        for c in range(NC1):
            pltpu.make_async_copy(
                w1_hbm.at[0, pl.ds(c * R1, R1), :],
                w1_buf.at[buf, pl.ds(c * R1, R1), :],
                sem1.at[buf, c]).wait()
        for c in range(NC2):
            pltpu.make_async_copy(
                w2_hbm.at[0, pl.ds(c * R2, R2), :],
                w2_buf.at[buf, pl.ds(c * R2, R2), :],
                sem2.at[buf, c]).wait()

    slot = jax.lax.rem(i, 2)
    nxt = jax.lax.rem(i + 1, 2)

    @pl.when(i == 0)
    def _prologue():
        start(0, 0)
        x = x_ref[...]
        ms = jnp.mean(x * x, axis=1, keepdims=True)
        xn = x * jax.lax.rsqrt(ms + EPS) * nw_ref[...]
        xn_ref[...] = xn
        g = jax.lax.dot_general(xn, gw_ref[...], (((1,), (1,)), ((), ())),
                                preferred_element_type=jnp.float32)
        g = g + gb_ref[...]
        iota = jax.lax.broadcasted_iota(jnp.int32, (T, E), 1)
        m1 = jnp.max(g, axis=1, keepdims=True)
        idx1 = jnp.min(jnp.where(g == m1, iota, E), axis=1, keepdims=True)
        g2 = jnp.where(iota == idx1, -jnp.inf, g)
        m2 = jnp.max(g2, axis=1, keepdims=True)
        idx2 = jnp.min(jnp.where(g2 == m2, iota, E), axis=1, keepdims=True)
        e2 = jnp.exp(m2 - m1)
        denom = 1.0 + e2
        wd_ref[...] = (jnp.where(iota == idx1, 1.0 / denom, 0.0)
                       + jnp.where(iota == idx2, e2 / denom, 0.0))
        out_ref[...] = x

    @pl.when(i + 1 < E)
    def _prefetch():
        start(i + 1, nxt)

    wait(slot)
    xn = xn_ref[...]
    xnb = xn.astype(jnp.bfloat16)
    w1 = w1_buf[slot].astype(jnp.bfloat16)
    b1g = b1g_ref[pl.ds(i, 1), :]
    b1l = b1l_ref[pl.ds(i, 1), :]
    hg = jax.lax.dot_general(xnb, w1[:, :DIM], (((1,), (1,)), ((), ())),
                             preferred_element_type=jnp.float32) + b1g
    hl = jax.lax.dot_general(xnb, w1[:, DIM:], (((1,), (1,)), ((), ())),
                             preferred_element_type=jnp.float32) + b1l
    hg = jnp.minimum(hg, LIMIT)
    hl = jnp.clip(hl, -LIMIT, LIMIT)
    act = hg * jax.nn.sigmoid(1.702 * hg) * (hl + 1.0)
    y = jax.lax.dot_general(act, w2_buf[slot], (((1,), (1,)), ((), ())),
                            preferred_element_type=jnp.float32)
    y = y + b2_ref[pl.ds(i, 1), :]
    iota = jax.lax.broadcasted_iota(jnp.int32, (T, E), 1)
    wcol = jnp.sum(jnp.where(iota == i, wd_ref[...], 0.0), axis=1,
                   keepdims=True)
    out_ref[...] += wcol * y


def kernel(x, freqs_cos, freqs_sin, gate_w, gate_b, w1, b1, w2, b2, norm_w):
    del freqs_cos, freqs_sin
    w1r = w1.reshape(E, HID, 2 * DIM)
    b1g = b1[:, 0::2]
    b1l = b1[:, 1::2]
    nw = norm_w.reshape(1, DIM)
    gb = gate_b.reshape(1, E)

    full = lambda shape: pl.BlockSpec(shape, lambda i: (0,) * len(shape))
    hbm = pl.BlockSpec(memory_space=pltpu.MemorySpace.HBM)

    return pl.pallas_call(
        _moe,
        grid=(E,),
        in_specs=[
            full((T, DIM)),
            full((1, DIM)),
            full((E, DIM)),
            full((1, E)),
            full((E, HID)),
            full((E, HID)),
            full((E, DIM)),
            hbm,
            hbm,
        ],
        out_specs=full((T, DIM)),
        out_shape=jax.ShapeDtypeStruct((T, DIM), jnp.float32),
        scratch_shapes=[
            pltpu.VMEM((T, DIM), jnp.float32),
            pltpu.VMEM((T, E), jnp.float32),
            pltpu.VMEM((2, HID, 2 * DIM), jnp.float32),
            pltpu.VMEM((2, DIM, HID), jnp.float32),
            pltpu.SemaphoreType.DMA((2, NC1)),
            pltpu.SemaphoreType.DMA((2, NC2)),
        ],
        compiler_params=pltpu.CompilerParams(
            dimension_semantics=("arbitrary",),
        ),
    )(x, nw, gate_w, gb, b1g, b1l, b2, w1r, w2)
